# Initial kernel scaffold; baseline (speedup 1.0000x reference)
import jax
import jax.numpy as jnp
from jax.experimental import pallas as pl


def kernel(x, segment_mask, pos_table, seg_table, gamma, beta):
    raise NotImplementedError("write your pallas kernel here")


# speedup vs baseline: 4.4090x; 4.4090x over previous
import jax
import jax.numpy as jnp
from jax.experimental import pallas as pl

_EPS = 1e-5
_TS = 512


def _embed_ln_kernel(x_ref, mask_ref, pos_ref, seg_ref, gamma_ref, beta_ref,
                     out_ref):
    x = x_ref[0]
    m = mask_ref[0]
    seg = jnp.where(m != 0, seg_ref[1:2, :], seg_ref[0:1, :])
    e = x + pos_ref[...] + seg
    mean = jnp.mean(e, axis=-1, keepdims=True)
    c = e - mean
    var = jnp.mean(c * c, axis=-1, keepdims=True)
    normed = c * jax.lax.rsqrt(var + _EPS)
    out_ref[0] = normed * gamma_ref[...] + beta_ref[...]


def kernel(x, segment_mask, pos_table, seg_table, gamma, beta):
    batch, seq, d = x.shape
    nb = seq // _TS
    mask3 = segment_mask.astype(jnp.int32).reshape(batch, seq, 1)
    gamma2 = gamma.reshape(1, d)
    beta2 = beta.reshape(1, d)
    return pl.pallas_call(
        _embed_ln_kernel,
        grid=(nb, batch),
        in_specs=[
            pl.BlockSpec((1, _TS, d), lambda n, b: (b, n, 0)),
            pl.BlockSpec((1, _TS, 1), lambda n, b: (b, n, 0)),
            pl.BlockSpec((_TS, d), lambda n, b: (n, 0)),
            pl.BlockSpec((2, d), lambda n, b: (0, 0)),
            pl.BlockSpec((1, d), lambda n, b: (0, 0)),
            pl.BlockSpec((1, d), lambda n, b: (0, 0)),
        ],
        out_specs=pl.BlockSpec((1, _TS, d), lambda n, b: (b, n, 0)),
        out_shape=jax.ShapeDtypeStruct((batch, seq, d), x.dtype),
    )(x, mask3, pos_table, seg_table, gamma2, beta2)


# speedup vs baseline: 4.8959x; 1.1104x over previous
import jax
import jax.numpy as jnp
from jax.experimental import pallas as pl

_EPS = 1e-5
_TS = 1024


def _embed_ln_kernel(x_ref, mask_ref, pos_ref, seg_ref, gamma_ref, beta_ref,
                     out_ref):
    x = x_ref[0]
    m = mask_ref[0]
    seg = jnp.where(m != 0, seg_ref[1:2, :], seg_ref[0:1, :])
    e = x + pos_ref[...] + seg
    mean = jnp.mean(e, axis=-1, keepdims=True)
    c = e - mean
    var = jnp.mean(c * c, axis=-1, keepdims=True)
    normed = c * jax.lax.rsqrt(var + _EPS)
    out_ref[0] = normed * gamma_ref[...] + beta_ref[...]


def kernel(x, segment_mask, pos_table, seg_table, gamma, beta):
    batch, seq, d = x.shape
    nb = seq // _TS
    mask3 = segment_mask.astype(jnp.int32).reshape(batch, seq, 1)
    gamma2 = gamma.reshape(1, d)
    beta2 = beta.reshape(1, d)
    return pl.pallas_call(
        _embed_ln_kernel,
        grid=(nb, batch),
        in_specs=[
            pl.BlockSpec((1, _TS, d), lambda n, b: (b, n, 0)),
            pl.BlockSpec((1, _TS, 1), lambda n, b: (b, n, 0)),
            pl.BlockSpec((_TS, d), lambda n, b: (n, 0)),
            pl.BlockSpec((2, d), lambda n, b: (0, 0)),
            pl.BlockSpec((1, d), lambda n, b: (0, 0)),
            pl.BlockSpec((1, d), lambda n, b: (0, 0)),
        ],
        out_specs=pl.BlockSpec((1, _TS, d), lambda n, b: (b, n, 0)),
        out_shape=jax.ShapeDtypeStruct((batch, seq, d), x.dtype),
    )(x, mask3, pos_table, seg_table, gamma2, beta2)


# speedup vs baseline: 5.0780x; 1.0372x over previous
import jax
import jax.numpy as jnp
from jax.experimental import pallas as pl
from jax.experimental.pallas import tpu as pltpu

_EPS = 1e-5
_TS = 2048


def _embed_ln_kernel(x_ref, mask_ref, pos_ref, seg_ref, gamma_ref, beta_ref,
                     out_ref):
    x = x_ref[0]
    m = mask_ref[0]
    seg = jnp.where(m != 0, seg_ref[1:2, :], seg_ref[0:1, :])
    e = x + pos_ref[...] + seg
    mean = jnp.mean(e, axis=-1, keepdims=True)
    c = e - mean
    var = jnp.mean(c * c, axis=-1, keepdims=True)
    normed = c * jax.lax.rsqrt(var + _EPS)
    out_ref[0] = normed * gamma_ref[...] + beta_ref[...]


def kernel(x, segment_mask, pos_table, seg_table, gamma, beta):
    batch, seq, d = x.shape
    nb = seq // _TS
    mask3 = segment_mask.astype(jnp.int32).reshape(batch, seq, 1)
    gamma2 = gamma.reshape(1, d)
    beta2 = beta.reshape(1, d)
    return pl.pallas_call(
        _embed_ln_kernel,
        grid=(nb, batch),
        in_specs=[
            pl.BlockSpec((1, _TS, d), lambda n, b: (b, n, 0)),
            pl.BlockSpec((1, _TS, 1), lambda n, b: (b, n, 0)),
            pl.BlockSpec((_TS, d), lambda n, b: (n, 0)),
            pl.BlockSpec((2, d), lambda n, b: (0, 0)),
            pl.BlockSpec((1, d), lambda n, b: (0, 0)),
            pl.BlockSpec((1, d), lambda n, b: (0, 0)),
        ],
        out_specs=pl.BlockSpec((1, _TS, d), lambda n, b: (b, n, 0)),
        out_shape=jax.ShapeDtypeStruct((batch, seq, d), x.dtype),
        compiler_params=pltpu.CompilerParams(
            vmem_limit_bytes=100 * 1024 * 1024),
    )(x, mask3, pos_table, seg_table, gamma2, beta2)


# speedup vs baseline: 5.0860x; 1.0016x over previous
import jax
import jax.numpy as jnp
from jax.experimental import pallas as pl
from jax.experimental.pallas import tpu as pltpu

_EPS = 1e-5
_TS = 2048


def _embed_ln_kernel(x_ref, mask_ref, pos_ref, seg_ref, gamma_ref, beta_ref,
                     out_ref):
    x = x_ref[0]
    m = mask_ref[0]
    seg = jnp.where(m != 0, seg_ref[1:2, :], seg_ref[0:1, :])
    e = x + pos_ref[...] + seg
    mean = jnp.mean(e, axis=-1, keepdims=True)
    c = e - mean
    var = jnp.mean(c * c, axis=-1, keepdims=True)
    normed = c * jax.lax.rsqrt(var + _EPS)
    out_ref[0] = normed * gamma_ref[...] + beta_ref[...]


def kernel(x, segment_mask, pos_table, seg_table, gamma, beta):
    batch, seq, d = x.shape
    nb = seq // _TS
    mask3 = segment_mask.astype(jnp.int32).reshape(batch, seq, 1)
    gamma2 = gamma.reshape(1, d)
    beta2 = beta.reshape(1, d)
    return pl.pallas_call(
        _embed_ln_kernel,
        grid=(nb, batch),
        in_specs=[
            pl.BlockSpec((1, _TS, d), lambda n, b: (b, n, 0)),
            pl.BlockSpec((1, _TS, 1), lambda n, b: (b, n, 0)),
            pl.BlockSpec((_TS, d), lambda n, b: (n, 0)),
            pl.BlockSpec((2, d), lambda n, b: (0, 0)),
            pl.BlockSpec((1, d), lambda n, b: (0, 0)),
            pl.BlockSpec((1, d), lambda n, b: (0, 0)),
        ],
        out_specs=pl.BlockSpec((1, _TS, d), lambda n, b: (b, n, 0)),
        out_shape=jax.ShapeDtypeStruct((batch, seq, d), x.dtype),
        compiler_params=pltpu.CompilerParams(
            vmem_limit_bytes=127 * 1024 * 1024),
    )(x, mask3, pos_table, seg_table, gamma2, beta2)
